# baseline (speedup 1.0000x reference)
import functools

import jax
import jax.numpy as jnp
from jax import lax
from jax.experimental import pallas as pl
from jax.experimental.pallas import tpu as pltpu


def _attend(x_ref, w_ref, H):
    BB, T, C = x_ref.shape
    x2d = x_ref[...].reshape(BB * T, C)
    kqv = jnp.dot(x2d, w_ref[...],
                  preferred_element_type=jnp.float32).reshape(BB, T, 3 * H)
    k = kqv[:, :, 0 * H:1 * H]
    q = kqv[:, :, 1 * H:2 * H]
    v = kqv[:, :, 2 * H:3 * H]

    wei = lax.dot_general(q, k, (((2,), (2,)), ((0,), (0,))),
                          preferred_element_type=jnp.float32)

    r = lax.broadcasted_iota(jnp.int32, (T, T), 0)
    c = lax.broadcasted_iota(jnp.int32, (T, T), 1)
    wei = jnp.where((c <= r)[None], wei, jnp.float32(-1e30))

    m = jnp.max(wei, axis=-1, keepdims=True)
    e = jnp.exp(wei - m)
    p = e / jnp.sum(e, axis=-1, keepdims=True)

    return lax.dot_general(p, v, (((2,), (1,)), ((0,), (0,))),
                           preferred_element_type=jnp.float32)


def _head_body(xa_ref, xb_ref, w_ref, o_ref, *, head_size):
    half = xa_ref.shape[0]
    o_ref[:half] = _attend(xa_ref, w_ref, head_size).astype(o_ref.dtype)
    o_ref[half:] = _attend(xb_ref, w_ref, head_size).astype(o_ref.dtype)


def kernel(x, wk, wq, wv):
    B, T, C = x.shape
    H = wk.shape[1]
    BB = 16
    half = BB // 2

    scale = float(C) ** -0.5
    w_kqv = jnp.concatenate([wk, wq * scale, wv], axis=1).astype(x.dtype)

    body = functools.partial(_head_body, head_size=H)
    return pl.pallas_call(
        body,
        out_shape=jax.ShapeDtypeStruct((B, T, H), x.dtype),
        grid=(B // BB,),
        in_specs=[
            pl.BlockSpec((half, T, C), lambda i: (2 * i, 0, 0)),
            pl.BlockSpec((half, T, C), lambda i: (2 * i + 1, 0, 0)),
            pl.BlockSpec((C, 3 * H), lambda i: (0, 0)),
        ],
        out_specs=pl.BlockSpec((BB, T, H), lambda i: (i, 0, 0)),
        compiler_params=pltpu.CompilerParams(
            dimension_semantics=("parallel",),
        ),
    )(x, x, w_kqv)


# speedup vs baseline: 1.1343x; 1.1343x over previous
import functools

import jax
import jax.numpy as jnp
from jax import lax
from jax.experimental import pallas as pl
from jax.experimental.pallas import tpu as pltpu


def _head_body(x_ref, w_ref, o_ref, *, head_size):
    H = head_size
    BB, T, C = x_ref.shape

    x2d = x_ref[...].reshape(BB * T, C)
    kqv = jnp.dot(x2d, w_ref[...],
                  preferred_element_type=jnp.float32).reshape(BB, T, 3 * H)
    k = kqv[:, :, 0 * H:1 * H]
    q = kqv[:, :, 1 * H:2 * H]
    v = kqv[:, :, 2 * H:3 * H]

    wei = lax.dot_general(q, k, (((2,), (2,)), ((0,), (0,))),
                          preferred_element_type=jnp.float32)

    r = lax.broadcasted_iota(jnp.int32, (T, T), 0)
    c = lax.broadcasted_iota(jnp.int32, (T, T), 1)
    wei = jnp.where((c <= r)[None], wei, jnp.float32(-1e30))

    m = jnp.max(wei, axis=-1, keepdims=True)
    e = jnp.exp(wei - m)
    p = e / jnp.sum(e, axis=-1, keepdims=True)

    out = lax.dot_general(p, v, (((2,), (1,)), ((0,), (0,))),
                          preferred_element_type=jnp.float32)
    o_ref[...] = out.astype(o_ref.dtype)


def kernel(x, wk, wq, wv):
    B, T, C = x.shape
    H = wk.shape[1]
    BB = 16

    scale = float(C) ** -0.5
    w_kqv = jnp.concatenate([wk, wq * scale, wv], axis=1).astype(x.dtype)

    body = functools.partial(_head_body, head_size=H)
    return pl.pallas_call(
        body,
        out_shape=jax.ShapeDtypeStruct((B, T, H), x.dtype),
        grid=(B // BB,),
        in_specs=[
            pl.BlockSpec((BB, T, C), lambda i: (i, 0, 0)),
            pl.BlockSpec((C, 3 * H), lambda i: (0, 0)),
        ],
        out_specs=pl.BlockSpec((BB, T, H), lambda i: (i, 0, 0)),
        compiler_params=pltpu.CompilerParams(
            dimension_semantics=("arbitrary",),
        ),
    )(x, w_kqv)
